# Initial kernel scaffold; baseline (speedup 1.0000x reference)
import functools

import jax
import jax.numpy as jnp
from jax import lax
from jax.experimental import pallas as pl
from jax.experimental.pallas import tpu as pltpu
from jax.experimental.pallas import tpu_sc as plsc

N = 10000
H = 10000
NNZ = 320000
D = 128

NC = 2
NS = 16
NW = NC * NS
L = 16

EPT = NNZ // NW
HP = 10240
CH = HP // NS
RH = HP // NW
NP_ = 10240
CN = NP_ // NS

IMAXV = jnp.int32(2**31 - 1)
IMINV = jnp.int32(-(2**31))
BIG = jnp.int32(N + 1)

F32 = jnp.float32
I32 = jnp.int32


def _mesh():
    return plsc.VectorSubcoreMesh(
        core_axis_name="c", subcore_axis_name="s", num_cores=NC, num_subcores=NS
    )


def _wid():
    return lax.axis_index("s") * NC + lax.axis_index("c")


def _iota():
    return lax.iota(I32, L)


def _sortable(pv):
    pi = plsc.bitcast(pv, I32)
    return jnp.where(pi < 0, pi ^ jnp.int32(0x7FFFFFFF), pi)


def _combine_runs(he_s, vals, ops, t_he, t_v):
    iota = _iota()
    for k in (1, 2, 4, 8):
        perm = jnp.maximum(iota - k, 0)
        t_he[...] = he_s
        he_sh = plsc.load_gather(t_he, [perm])
        eq = (he_sh == he_s) & (iota >= k)
        new_vals = []
        for v, op in zip(vals, ops):
            t_v[...] = v
            v_sh = plsc.load_gather(t_v, [perm])
            new_vals.append(jnp.where(eq, op(v, v_sh), v))
        vals = new_vals
    up = jnp.minimum(iota + 1, L - 1)
    t_he[...] = he_s
    he_nx = plsc.load_gather(t_he, [up])
    is_last = (he_nx != he_s) | (iota == L - 1)
    return vals, is_last


def _sc_reduce_to_hbm(priv_refs, shared_refs, red_v, outbuf, out_hbm_parts, opfns):
    c = lax.axis_index("c")
    s = lax.axis_index("s")
    for priv, sh in zip(priv_refs, shared_refs):
        pltpu.sync_copy(priv, sh.at[s])
    plsc.subcore_barrier()
    col = s * CH
    for sh, out_hbm, opfn in zip(shared_refs, out_hbm_parts, opfns):
        for r in range(NS):
            pltpu.sync_copy(sh.at[r, pl.ds(col, CH)], red_v.at[r])

        def red_body(j, _, opfn=opfn):
            sl = pl.ds(j * L, L)
            acc = red_v[0, sl]
            for r in range(1, NS):
                acc = opfn(acc, red_v[r, sl])
            outbuf[sl] = acc
            return None

        lax.fori_loop(0, CH // L, red_body, None)
        pltpu.sync_copy(outbuf, out_hbm.at[c, pl.ds(col, CH)])


_K1_BLK = 400
_K1_GRID = N // _K1_BLK


def _k1_body(x_ref, w_ref, b_ref, d_ref, xt_ref, p_ref):
    xb = x_ref[...]
    xt = lax.dot_general(
        xb, w_ref[...], (((1,), (1,)), ((), ())),
        preferred_element_type=F32, precision=lax.Precision.HIGHEST,
    )
    xt = xt + b_ref[...]
    xt_ref[...] = xt
    p_ref[...] = lax.dot_general(
        d_ref[...], xt, (((1,), (1,)), ((), ())),
        preferred_element_type=F32, precision=lax.Precision.HIGHEST,
    )


def _k1(x, w, b2, d2):
    return pl.pallas_call(
        _k1_body,
        grid=(_K1_GRID,),
        in_specs=[
            pl.BlockSpec((_K1_BLK, D), lambda i: (i, 0)),
            pl.BlockSpec((D, D), lambda i: (0, 0)),
            pl.BlockSpec((1, D), lambda i: (0, 0)),
            pl.BlockSpec((1, D), lambda i: (0, 0)),
        ],
        out_specs=[
            pl.BlockSpec((_K1_BLK, D), lambda i: (i, 0)),
            pl.BlockSpec((1, _K1_BLK), lambda i: (i, 0)),
        ],
        out_shape=[
            jax.ShapeDtypeStruct((N, D), F32),
            jax.ShapeDtypeStruct((_K1_GRID, _K1_BLK), F32),
        ],
    )(x, w, b2, d2)


def _k2(p, nodes, he):
    @functools.partial(
        pl.kernel,
        mesh=_mesh(),
        out_type=(
            jax.ShapeDtypeStruct((NC, HP), I32),
            jax.ShapeDtypeStruct((NC, HP), I32),
        ),
        scratch_types=[
            pltpu.VMEM((N,), F32),
            pltpu.VMEM((EPT,), I32),
            pltpu.VMEM((EPT,), I32),
            pltpu.VMEM((HP,), I32),
            pltpu.VMEM((HP,), I32),
            pltpu.VMEM_SHARED((NS, HP), I32),
            pltpu.VMEM_SHARED((NS, HP), I32),
            pltpu.VMEM((NS, CH), I32),
            pltpu.VMEM((CH,), I32),
            pltpu.VMEM((L,), I32),
            pltpu.VMEM((L,), I32),
        ],
    )
    def k2(p_hbm, nodes_hbm, he_hbm, smax_hbm, smin_hbm,
           p_v, nd_v, he_v, pmax, pmin, shmax, shmin, red_v, outbuf, t_he, t_v):
        wid = _wid()
        base = wid * EPT
        pltpu.sync_copy(p_hbm, p_v)
        pltpu.sync_copy(nodes_hbm.at[pl.ds(base, EPT)], nd_v)
        pltpu.sync_copy(he_hbm.at[pl.ds(base, EPT)], he_v)

        def init_body(i, _):
            sl = pl.ds(i * L, L)
            pmax[sl] = jnp.full((L,), IMINV, I32)
            pmin[sl] = jnp.full((L,), IMAXV, I32)
            return None

        lax.fori_loop(0, HP // L, init_body, None)

        def body(i, _):
            sl = pl.ds(i * L, L)
            nd = nd_v[sl]
            hv = he_v[sl]
            key = _sortable(plsc.load_gather(p_v, [nd]))
            he_s, key_s = plsc.sort_key_val(hv, key)
            (vmax, vmin), is_last = _combine_runs(
                he_s, [key_s, key_s], [jnp.maximum, jnp.minimum], t_he, t_v)
            cur = plsc.load_gather(pmax, [he_s])
            plsc.store_scatter(pmax, [he_s], jnp.maximum(cur, vmax), mask=is_last)
            cur = plsc.load_gather(pmin, [he_s])
            plsc.store_scatter(pmin, [he_s], jnp.minimum(cur, vmin), mask=is_last)
            return None

        lax.fori_loop(0, EPT // L, body, None)

        _sc_reduce_to_hbm([pmax, pmin], [shmax, shmin], red_v, outbuf,
                          [smax_hbm, smin_hbm], [jnp.maximum, jnp.minimum])

    return k2(p, nodes, he)


def _k4(p, nodes, he, smax2, smin2):
    @functools.partial(
        pl.kernel,
        mesh=_mesh(),
        out_type=(
            jax.ShapeDtypeStruct((NC, HP), I32),
            jax.ShapeDtypeStruct((NC, HP), I32),
        ),
        scratch_types=[
            pltpu.VMEM((N,), F32),
            pltpu.VMEM((EPT,), I32),
            pltpu.VMEM((EPT,), I32),
            pltpu.VMEM((HP,), I32),
            pltpu.VMEM((HP,), I32),
            pltpu.VMEM((HP,), I32),
            pltpu.VMEM((HP,), I32),
            pltpu.VMEM((HP,), I32),
            pltpu.VMEM_SHARED((NS, HP), I32),
            pltpu.VMEM_SHARED((NS, HP), I32),
            pltpu.VMEM((NS, CH), I32),
            pltpu.VMEM((CH,), I32),
            pltpu.VMEM((L,), I32),
            pltpu.VMEM((L,), I32),
        ],
    )
    def k4(p_hbm, nodes_hbm, he_hbm, smax_hbm, smin_hbm, ia_hbm, ja_hbm,
           p_v, nd_v, he_v, smax, smin, tmp, pi_e, pj_e,
           shi, shj, red_v, outbuf, t_he, t_v):
        wid = _wid()
        base = wid * EPT
        pltpu.sync_copy(p_hbm, p_v)
        pltpu.sync_copy(nodes_hbm.at[pl.ds(base, EPT)], nd_v)
        pltpu.sync_copy(he_hbm.at[pl.ds(base, EPT)], he_v)
        pltpu.sync_copy(smax_hbm.at[0], smax)
        pltpu.sync_copy(smax_hbm.at[1], tmp)

        def comb_max(i, _):
            sl = pl.ds(i * L, L)
            smax[sl] = jnp.maximum(smax[sl], tmp[sl])
            return None

        lax.fori_loop(0, HP // L, comb_max, None)
        pltpu.sync_copy(smin_hbm.at[0], smin)
        pltpu.sync_copy(smin_hbm.at[1], tmp)

        def comb_min(i, _):
            sl = pl.ds(i * L, L)
            smin[sl] = jnp.minimum(smin[sl], tmp[sl])
            pi_e[sl] = jnp.full((L,), IMAXV, I32)
            pj_e[sl] = jnp.full((L,), IMAXV, I32)
            return None

        lax.fori_loop(0, HP // L, comb_min, None)

        def body(i, _):
            sl = pl.ds(i * L, L)
            nd = nd_v[sl]
            hv = he_v[sl]
            key = _sortable(plsc.load_gather(p_v, [nd]))
            gmax = plsc.load_gather(smax, [hv])
            gmin = plsc.load_gather(smin, [hv])
            ci = jnp.where(key >= gmax, nd, BIG)
            cj = jnp.where(key <= gmin, nd, BIG)
            he_s, ci_s = plsc.sort_key_val(hv, ci)
            _, cj_s = plsc.sort_key_val(hv, cj)
            (vi, vj), is_last = _combine_runs(
                he_s, [ci_s, cj_s], [jnp.minimum, jnp.minimum], t_he, t_v)
            cur = plsc.load_gather(pi_e, [he_s])
            plsc.store_scatter(pi_e, [he_s], jnp.minimum(cur, vi), mask=is_last)
            cur = plsc.load_gather(pj_e, [he_s])
            plsc.store_scatter(pj_e, [he_s], jnp.minimum(cur, vj), mask=is_last)
            return None

        lax.fori_loop(0, EPT // L, body, None)

        _sc_reduce_to_hbm([pi_e, pj_e], [shi, shj], red_v, outbuf,
                          [ia_hbm, ja_hbm], [jnp.minimum, jnp.minimum])

    return k4(p, nodes, he, smax2, smin2)


_NCH = RH // 80


def _k6a(ia2, ja2):
    @functools.partial(
        pl.kernel,
        mesh=_mesh(),
        out_type=(
            jax.ShapeDtypeStruct((NW, _NCH, 80), I32),
            jax.ShapeDtypeStruct((NW, _NCH, 80), I32),
            jax.ShapeDtypeStruct((NW, _NCH, 80), F32),
            jax.ShapeDtypeStruct((NC, NP_), F32),
        ),
        scratch_types=[
            pltpu.VMEM((RH,), I32),
            pltpu.VMEM((RH,), I32),
            pltpu.VMEM((RH,), I32),
            pltpu.VMEM((RH,), I32),
            pltpu.VMEM((_NCH, 80), I32),
            pltpu.VMEM((_NCH, 80), I32),
            pltpu.VMEM((_NCH, 80), F32),
            pltpu.VMEM((CN,), F32),
            pltpu.VMEM_SHARED((NP_,), F32),
        ],
    )
    def k6a(ia_hbm, ja_hbm, ie_hbm, je_hbm, w_hbm, degp_hbm,
            a0, a1, b0, b1, ie_idx, je_idx, w_v, zbuf, deg_sp):
        c = lax.axis_index("c")
        s = lax.axis_index("s")
        wid = _wid()
        hbase = wid * RH
        pltpu.sync_copy(ia_hbm.at[0, pl.ds(hbase, RH)], a0)
        pltpu.sync_copy(ia_hbm.at[1, pl.ds(hbase, RH)], a1)
        pltpu.sync_copy(ja_hbm.at[0, pl.ds(hbase, RH)], b0)
        pltpu.sync_copy(ja_hbm.at[1, pl.ds(hbase, RH)], b1)
        for i in range(RH // L):
            sl = pl.ds(i * L, L)
            ie = jnp.minimum(a0[sl], a1[sl])
            je = jnp.minimum(b0[sl], b1[sl])
            valid = (ie < N) & (je < N)
            w = jnp.where(valid, jnp.float32(1.0), jnp.float32(0.0))
            ie = jnp.where(valid, ie, 0)
            je = jnp.where(valid, je, 0)
            ci, ro = divmod(i, 5)
            sl2 = pl.ds(ro * L, L)
            ie_idx[ci, sl2] = ie
            je_idx[ci, sl2] = je
            w_v[ci, sl2] = w
        pltpu.sync_copy(ie_idx, ie_hbm.at[wid])
        pltpu.sync_copy(je_idx, je_hbm.at[wid])
        pltpu.sync_copy(w_v, w_hbm.at[wid])

        def zero_body(i, _):
            zbuf[pl.ds(i * L, L)] = jnp.zeros((L,), F32)
            return None

        lax.fori_loop(0, CN // L, zero_body, None)
        pltpu.sync_copy(zbuf, deg_sp.at[pl.ds(s * CN, CN)])
        plsc.subcore_barrier()
        for ci in range(_NCH):
            pltpu.sync_copy(w_v.at[ci], deg_sp.at[ie_idx.at[ci]], add=True)
            pltpu.sync_copy(w_v.at[ci], deg_sp.at[je_idx.at[ci]], add=True)
        plsc.subcore_barrier()
        pltpu.sync_copy(deg_sp.at[pl.ds(s * CN, CN)],
                        degp_hbm.at[c, pl.ds(s * CN, CN)])

    return k6a(ia2, ja2)


def _k6b(xt, degp, ie3, je3, w3):
    @functools.partial(
        pl.kernel,
        mesh=_mesh(),
        out_type=jax.ShapeDtypeStruct((NC, NP_, D), F32),
        scratch_types=[
            pltpu.VMEM((NP_,), F32),
            pltpu.VMEM((NP_,), F32),
            pltpu.VMEM((_NCH, 80), I32),
            pltpu.VMEM((_NCH, 80), I32),
            pltpu.VMEM((_NCH, 80), F32),
            pltpu.VMEM((_NCH, 80), F32),
            pltpu.VMEM((80, D), F32),
            pltpu.VMEM((128, D), F32),
            pltpu.VMEM_SHARED((NP_, D), F32),
        ],
    )
    def k6b(xt_hbm, degp_hbm, ie_hbm, je_hbm, w_hbm, outp_hbm,
            tmp, dinv, ie_idx, je_idx, w_v, coef_v, rows, zblk, out_sp):
        c = lax.axis_index("c")
        s = lax.axis_index("s")
        wid = _wid()

        def zfill(i, _):
            for q in range(D // L):
                zblk[i, pl.ds(q * L, L)] = jnp.zeros((L,), F32)
            return None

        lax.fori_loop(0, 128, zfill, None)
        for kk in range(CN // 128):
            pltpu.sync_copy(zblk, out_sp.at[pl.ds(s * CN + kk * 128, 128)])

        pltpu.sync_copy(degp_hbm.at[0], tmp)
        pltpu.sync_copy(degp_hbm.at[1], dinv)

        def dinv_body(i, _):
            sl = pl.ds(i * L, L)
            d = tmp[sl] + dinv[sl] + jnp.float32(1.0)
            bi = plsc.bitcast(d, I32)
            y = plsc.bitcast(jnp.int32(0x5F3759DF) - (bi >> 1), F32)
            half_d = jnp.float32(0.5) * d
            for _u in range(3):
                y = y * (jnp.float32(1.5) - half_d * y * y)
            dinv[sl] = y
            return None

        lax.fori_loop(0, NP_ // L, dinv_body, None)

        pltpu.sync_copy(ie_hbm.at[wid], ie_idx)
        pltpu.sync_copy(je_hbm.at[wid], je_idx)
        pltpu.sync_copy(w_hbm.at[wid], w_v)
        for i in range(RH // L):
            ci, ro = divmod(i, 5)
            sl = pl.ds(ro * L, L)
            di = plsc.load_gather(dinv, [ie_idx[ci, sl]])
            dj = plsc.load_gather(dinv, [je_idx[ci, sl]])
            coef_v[ci, sl] = w_v[ci, sl] * di * dj
        plsc.subcore_barrier()

        def scale_body(e, ci):
            cf = coef_v[ci, e]
            for q in range(D // L):
                sl = pl.ds(q * L, L)
                rows[e, sl] = rows[e, sl] * cf
            return ci

        for ci in range(_NCH):
            pltpu.sync_copy(xt_hbm.at[je_idx.at[ci]], rows)
            lax.fori_loop(0, 80, scale_body, ci)
            pltpu.sync_copy(rows, out_sp.at[ie_idx.at[ci]], add=True)
            pltpu.sync_copy(xt_hbm.at[ie_idx.at[ci]], rows)
            lax.fori_loop(0, 80, scale_body, ci)
            pltpu.sync_copy(rows, out_sp.at[je_idx.at[ci]], add=True)
        plsc.subcore_barrier()
        pltpu.sync_copy(out_sp.at[pl.ds(s * CN, CN)],
                        outp_hbm.at[c, pl.ds(s * CN, CN)])

    return k6b(xt, degp, ie3, je3, w3)


def _k8_body(xt_ref, dg_ref, o0_ref, o1_ref, out_ref):
    d = jnp.float32(1.0) + dg_ref[:, 0:1] + dg_ref[:, 1:2]
    inv = jnp.float32(1.0) / d
    out_ref[...] = jnp.maximum(xt_ref[...] * inv + o0_ref[...] + o1_ref[...],
                               jnp.float32(0.0))


def _k8(xt, degT, o0, o1):
    return pl.pallas_call(
        _k8_body,
        grid=(_K1_GRID,),
        in_specs=[
            pl.BlockSpec((_K1_BLK, D), lambda i: (i, 0)),
            pl.BlockSpec((_K1_BLK, NC), lambda i: (i, 0)),
            pl.BlockSpec((_K1_BLK, D), lambda i: (i, 0)),
            pl.BlockSpec((_K1_BLK, D), lambda i: (i, 0)),
        ],
        out_specs=pl.BlockSpec((_K1_BLK, D), lambda i: (i, 0)),
        out_shape=jax.ShapeDtypeStruct((N, D), F32),
    )(xt, degT, o0, o1)


def kernel(x, hyperedge_index, W, b, direction):
    nodes = hyperedge_index[0]
    he = hyperedge_index[1]
    b2 = b.reshape(1, D)
    d2 = direction.reshape(1, D)
    xt, p2 = _k1(x, W, b2, d2)
    p = p2.reshape(N)
    smax2, smin2 = _k2(p, nodes, he)
    ia2, ja2 = _k4(p, nodes, he, smax2, smin2)
    ie3, je3, w3, degp = _k6a(ia2, ja2)
    outp = _k6b(xt, degp, ie3, je3, w3)
    degT = jnp.stack([degp[0, :N], degp[1, :N]], axis=-1)
    out = _k8(xt, degT, outp[0, :N], outp[1, :N])
    return out


# speedup vs baseline: 26.4902x; 26.4902x over previous
import functools

import jax
import jax.numpy as jnp
from jax import lax
from jax.experimental import pallas as pl
from jax.experimental.pallas import tpu as pltpu
from jax.experimental.pallas import tpu_sc as plsc

N = 10000
H = 10000
NNZ = 320000
D = 128

NC = 2
NS = 16
NW = NC * NS
L = 16

EPT = NNZ // NW
HP = 10240
CH = HP // NS
RH = HP // NW
NP_ = 10240
CN = NP_ // NS

IMAXV = 2**31 - 1
IMINV = -(2**31)
BIG = N + 1

F32 = jnp.float32
I32 = jnp.int32


def _mesh():
    return plsc.VectorSubcoreMesh(
        core_axis_name="c", subcore_axis_name="s", num_cores=NC, num_subcores=NS
    )


_SC_PARAMS = pltpu.CompilerParams(
    use_tc_tiling_on_sc=False, needs_layout_passes=False
)


def _wid():
    return lax.axis_index("s") * NC + lax.axis_index("c")


def _iota():
    return lax.iota(I32, L)


def _sortable(pv):
    pi = plsc.bitcast(pv, I32)
    return jnp.where(pi < 0, pi ^ jnp.int32(0x7FFFFFFF), pi)


def _combine_runs(he_s, vals, ops, t_he, t_v):
    iota = _iota()
    for k in (1, 2, 4, 8):
        perm = jnp.maximum(iota - k, 0)
        t_he[pl.ds(0, L)] = he_s
        he_sh = plsc.load_gather(t_he, [perm])
        eq = (he_sh == he_s) & (iota >= k)
        new_vals = []
        for v, op in zip(vals, ops):
            t_v[pl.ds(0, L)] = v
            v_sh = plsc.load_gather(t_v, [perm])
            new_vals.append(jnp.where(eq, op(v, v_sh), v))
        vals = new_vals
    up = jnp.minimum(iota + 1, L - 1)
    t_he[pl.ds(0, L)] = he_s
    he_nx = plsc.load_gather(t_he, [up])
    is_last = (he_nx != he_s) | (iota == L - 1)
    return vals, is_last


def _sc_reduce_to_hbm(priv_refs, shared_refs, red_v, outbuf, out_hbm_parts, opfns):
    c = lax.axis_index("c")
    s = lax.axis_index("s")
    for priv, sh in zip(priv_refs, shared_refs):
        pltpu.sync_copy(priv, sh.at[s])
    plsc.subcore_barrier()
    col = s * CH
    for sh, out_hbm, opfn in zip(shared_refs, out_hbm_parts, opfns):
        for r in range(NS):
            pltpu.sync_copy(sh.at[r, pl.ds(col, CH)], red_v.at[r])

        def red_body(j, _, opfn=opfn):
            sl = pl.ds(j * L, L)
            acc = red_v[0, sl]
            for r in range(1, NS):
                acc = opfn(acc, red_v[r, sl])
            outbuf[sl] = acc
            return None

        lax.fori_loop(0, CH // L, red_body, None)
        pltpu.sync_copy(outbuf, out_hbm.at[c, pl.ds(col, CH)])


_K1_BLK = 400
_K1_GRID = N // _K1_BLK


def _k1_body(x_ref, w_ref, b_ref, d_ref, xt_ref, p_ref):
    xb = x_ref[...]
    xt = lax.dot_general(
        xb, w_ref[...], (((1,), (1,)), ((), ())),
        preferred_element_type=F32,
    )
    xt = xt + b_ref[...]
    xt_ref[...] = xt
    p_ref[...] = lax.dot_general(
        d_ref[...], xt, (((1,), (1,)), ((), ())),
        preferred_element_type=F32,
    ).reshape(1, 1, _K1_BLK)


def _k1(x, w, b2, d2):
    return pl.pallas_call(
        _k1_body,
        grid=(_K1_GRID,),
        in_specs=[
            pl.BlockSpec((_K1_BLK, D), lambda i: (i, 0)),
            pl.BlockSpec((D, D), lambda i: (0, 0)),
            pl.BlockSpec((1, D), lambda i: (0, 0)),
            pl.BlockSpec((1, D), lambda i: (0, 0)),
        ],
        out_specs=[
            pl.BlockSpec((_K1_BLK, D), lambda i: (i, 0)),
            pl.BlockSpec((1, 1, _K1_BLK), lambda i: (i, 0, 0)),
        ],
        out_shape=[
            jax.ShapeDtypeStruct((N, D), F32),
            jax.ShapeDtypeStruct((_K1_GRID, 1, _K1_BLK), F32),
        ],
    )(x, w, b2, d2)


def _k2(p, nodes, he):
    @functools.partial(
        pl.kernel,
        mesh=_mesh(),
        compiler_params=_SC_PARAMS,
        out_type=(
            jax.ShapeDtypeStruct((NC, HP), I32),
            jax.ShapeDtypeStruct((NC, HP), I32),
        ),
        scratch_types=[
            pltpu.VMEM((HP,), F32),
            pltpu.VMEM((EPT,), I32),
            pltpu.VMEM((EPT,), I32),
            pltpu.VMEM((HP,), I32),
            pltpu.VMEM((HP,), I32),
            pltpu.VMEM_SHARED((NS, HP), I32),
            pltpu.VMEM_SHARED((NS, HP), I32),
            pltpu.VMEM((NS, CH), I32),
            pltpu.VMEM((CH,), I32),
            pltpu.VMEM((128,), I32),
            pltpu.VMEM((128,), I32),
        ],
    )
    def k2(p_hbm, nodes_hbm, he_hbm, smax_hbm, smin_hbm,
           p_v, nd_v, he_v, pmax, pmin, shmax, shmin, red_v, outbuf, t_he, t_v):
        wid = _wid()
        base = wid * EPT
        pltpu.sync_copy(p_hbm, p_v)
        pltpu.sync_copy(nodes_hbm.at[pl.ds(base, EPT)], nd_v)
        pltpu.sync_copy(he_hbm.at[pl.ds(base, EPT)], he_v)

        def init_body(i, _):
            sl = pl.ds(i * L, L)
            pmax[sl] = jnp.full((L,), IMINV, I32)
            pmin[sl] = jnp.full((L,), IMAXV, I32)
            return None

        lax.fori_loop(0, HP // L, init_body, None)

        def body(i, _):
            sl = pl.ds(i * L, L)
            nd = nd_v[sl]
            hv = he_v[sl]
            key = _sortable(plsc.load_gather(p_v, [nd]))
            he_s, key_s = plsc.sort_key_val(hv, key)
            (vmax, vmin), is_last = _combine_runs(
                he_s, [key_s, key_s], [jnp.maximum, jnp.minimum], t_he, t_v)
            cur = plsc.load_gather(pmax, [he_s])
            plsc.store_scatter(pmax, [he_s], jnp.maximum(cur, vmax), mask=is_last)
            cur = plsc.load_gather(pmin, [he_s])
            plsc.store_scatter(pmin, [he_s], jnp.minimum(cur, vmin), mask=is_last)
            return None

        lax.fori_loop(0, EPT // L, body, None)

        _sc_reduce_to_hbm([pmax, pmin], [shmax, shmin], red_v, outbuf,
                          [smax_hbm, smin_hbm], [jnp.maximum, jnp.minimum])

    return k2(p, nodes, he)


def _k4(p, nodes, he, smax2, smin2):
    @functools.partial(
        pl.kernel,
        mesh=_mesh(),
        compiler_params=_SC_PARAMS,
        out_type=(
            jax.ShapeDtypeStruct((NC, HP), I32),
            jax.ShapeDtypeStruct((NC, HP), I32),
        ),
        scratch_types=[
            pltpu.VMEM((HP,), F32),
            pltpu.VMEM((EPT,), I32),
            pltpu.VMEM((EPT,), I32),
            pltpu.VMEM((HP,), I32),
            pltpu.VMEM((HP,), I32),
            pltpu.VMEM((HP,), I32),
            pltpu.VMEM((HP,), I32),
            pltpu.VMEM((HP,), I32),
            pltpu.VMEM_SHARED((NS, HP), I32),
            pltpu.VMEM_SHARED((NS, HP), I32),
            pltpu.VMEM((NS, CH), I32),
            pltpu.VMEM((CH,), I32),
            pltpu.VMEM((128,), I32),
            pltpu.VMEM((128,), I32),
        ],
    )
    def k4(p_hbm, nodes_hbm, he_hbm, smax_hbm, smin_hbm, ia_hbm, ja_hbm,
           p_v, nd_v, he_v, smax, smin, tmp, pi_e, pj_e,
           shi, shj, red_v, outbuf, t_he, t_v):
        wid = _wid()
        base = wid * EPT
        pltpu.sync_copy(p_hbm, p_v)
        pltpu.sync_copy(nodes_hbm.at[pl.ds(base, EPT)], nd_v)
        pltpu.sync_copy(he_hbm.at[pl.ds(base, EPT)], he_v)
        pltpu.sync_copy(smax_hbm.at[0], smax)
        pltpu.sync_copy(smax_hbm.at[1], tmp)

        def comb_max(i, _):
            sl = pl.ds(i * L, L)
            smax[sl] = jnp.maximum(smax[sl], tmp[sl])
            return None

        lax.fori_loop(0, HP // L, comb_max, None)
        pltpu.sync_copy(smin_hbm.at[0], smin)
        pltpu.sync_copy(smin_hbm.at[1], tmp)

        def comb_min(i, _):
            sl = pl.ds(i * L, L)
            smin[sl] = jnp.minimum(smin[sl], tmp[sl])
            pi_e[sl] = jnp.full((L,), IMAXV, I32)
            pj_e[sl] = jnp.full((L,), IMAXV, I32)
            return None

        lax.fori_loop(0, HP // L, comb_min, None)

        def body(i, _):
            sl = pl.ds(i * L, L)
            nd = nd_v[sl]
            hv = he_v[sl]
            key = _sortable(plsc.load_gather(p_v, [nd]))
            gmax = plsc.load_gather(smax, [hv])
            gmin = plsc.load_gather(smin, [hv])
            ci = jnp.where(key >= gmax, nd, BIG)
            cj = jnp.where(key <= gmin, nd, BIG)
            he_s, ci_s = plsc.sort_key_val(hv, ci)
            _, cj_s = plsc.sort_key_val(hv, cj)
            (vi, vj), is_last = _combine_runs(
                he_s, [ci_s, cj_s], [jnp.minimum, jnp.minimum], t_he, t_v)
            cur = plsc.load_gather(pi_e, [he_s])
            plsc.store_scatter(pi_e, [he_s], jnp.minimum(cur, vi), mask=is_last)
            cur = plsc.load_gather(pj_e, [he_s])
            plsc.store_scatter(pj_e, [he_s], jnp.minimum(cur, vj), mask=is_last)
            return None

        lax.fori_loop(0, EPT // L, body, None)

        _sc_reduce_to_hbm([pi_e, pj_e], [shi, shj], red_v, outbuf,
                          [ia_hbm, ja_hbm], [jnp.minimum, jnp.minimum])

    return k4(p, nodes, he, smax2, smin2)


_NCH = RH // 80


def _k6a(ia2, ja2):
    @functools.partial(
        pl.kernel,
        mesh=_mesh(),
        compiler_params=_SC_PARAMS,
        out_type=(
            jax.ShapeDtypeStruct((NW, _NCH, 80), I32),
            jax.ShapeDtypeStruct((NW, _NCH, 80), I32),
            jax.ShapeDtypeStruct((NW, _NCH, 80), F32),
            jax.ShapeDtypeStruct((NC, NP_), F32),
        ),
        scratch_types=[
            pltpu.VMEM((RH,), I32),
            pltpu.VMEM((RH,), I32),
            pltpu.VMEM((RH,), I32),
            pltpu.VMEM((RH,), I32),
            pltpu.VMEM((_NCH, 80), I32),
            pltpu.VMEM((_NCH, 80), I32),
            pltpu.VMEM((_NCH, 80), F32),
            pltpu.VMEM((CN,), F32),
            pltpu.VMEM_SHARED((NP_,), F32),
        ],
    )
    def k6a(ia_hbm, ja_hbm, ie_hbm, je_hbm, w_hbm, degp_hbm,
            a0, a1, b0, b1, ie_idx, je_idx, w_v, zbuf, deg_sp):
        c = lax.axis_index("c")
        s = lax.axis_index("s")
        wid = _wid()
        hbase = wid * RH
        pltpu.sync_copy(ia_hbm.at[0, pl.ds(hbase, RH)], a0)
        pltpu.sync_copy(ia_hbm.at[1, pl.ds(hbase, RH)], a1)
        pltpu.sync_copy(ja_hbm.at[0, pl.ds(hbase, RH)], b0)
        pltpu.sync_copy(ja_hbm.at[1, pl.ds(hbase, RH)], b1)
        for i in range(RH // L):
            sl = pl.ds(i * L, L)
            ie = jnp.minimum(a0[sl], a1[sl])
            je = jnp.minimum(b0[sl], b1[sl])
            valid = (ie < N) & (je < N)
            w = jnp.where(valid, jnp.float32(1.0), jnp.float32(0.0))
            ie = jnp.where(valid, ie, 0)
            je = jnp.where(valid, je, 0)
            ci, ro = divmod(i, 5)
            sl2 = pl.ds(ro * L, L)
            ie_idx[ci, sl2] = ie
            je_idx[ci, sl2] = je
            w_v[ci, sl2] = w
        pltpu.sync_copy(ie_idx, ie_hbm.at[wid])
        pltpu.sync_copy(je_idx, je_hbm.at[wid])
        pltpu.sync_copy(w_v, w_hbm.at[wid])

        def zero_body(i, _):
            zbuf[pl.ds(i * L, L)] = jnp.zeros((L,), F32)
            return None

        lax.fori_loop(0, CN // L, zero_body, None)
        pltpu.sync_copy(zbuf, deg_sp.at[pl.ds(s * CN, CN)])
        plsc.subcore_barrier()
        for ci in range(_NCH):
            pltpu.sync_copy(w_v.at[ci], deg_sp.at[ie_idx.at[ci]], add=True)
            pltpu.sync_copy(w_v.at[ci], deg_sp.at[je_idx.at[ci]], add=True)
        plsc.subcore_barrier()
        pltpu.sync_copy(deg_sp.at[pl.ds(s * CN, CN)],
                        degp_hbm.at[c, pl.ds(s * CN, CN)])

    return k6a(ia2, ja2)


def _k6b(xt, degp, ie3, je3, w3):
    @functools.partial(
        pl.kernel,
        mesh=_mesh(),
        compiler_params=_SC_PARAMS,
        out_type=jax.ShapeDtypeStruct((NC, NP_, D), F32),
        scratch_types=[
            pltpu.VMEM((NP_,), F32),
            pltpu.VMEM((NP_,), F32),
            pltpu.VMEM((_NCH, 80), I32),
            pltpu.VMEM((_NCH, 80), I32),
            pltpu.VMEM((_NCH, 80), F32),
            pltpu.VMEM((_NCH, 80), F32),
            pltpu.VMEM((80, D), F32),
            pltpu.VMEM((128, D), F32),
            pltpu.VMEM((128,), F32),
            pltpu.VMEM_SHARED((NP_, D), F32),
        ],
    )
    def k6b(xt_hbm, degp_hbm, ie_hbm, je_hbm, w_hbm, outp_hbm,
            tmp, dinv, ie_idx, je_idx, w_v, coef_v, rows, zblk, t_c, out_sp):
        c = lax.axis_index("c")
        s = lax.axis_index("s")
        wid = _wid()

        def zfill(i, _):
            for q in range(D // L):
                zblk[i, pl.ds(q * L, L)] = jnp.zeros((L,), F32)
            return None

        lax.fori_loop(0, 128, zfill, None)
        for kk in range(CN // 128):
            pltpu.sync_copy(zblk, out_sp.at[pl.ds(s * CN + kk * 128, 128)])

        pltpu.sync_copy(degp_hbm.at[0], tmp)
        pltpu.sync_copy(degp_hbm.at[1], dinv)

        def dinv_body(i, _):
            sl = pl.ds(i * L, L)
            d = tmp[sl] + dinv[sl] + jnp.float32(1.0)
            bi = plsc.bitcast(d, I32)
            y = plsc.bitcast(jnp.int32(0x5F3759DF) - (bi >> 1), F32)
            half_d = jnp.float32(0.5) * d
            for _u in range(3):
                y = y * (jnp.float32(1.5) - half_d * y * y)
            dinv[sl] = y
            return None

        lax.fori_loop(0, NP_ // L, dinv_body, None)

        pltpu.sync_copy(ie_hbm.at[wid], ie_idx)
        pltpu.sync_copy(je_hbm.at[wid], je_idx)
        pltpu.sync_copy(w_hbm.at[wid], w_v)
        for i in range(RH // L):
            ci, ro = divmod(i, 5)
            sl = pl.ds(ro * L, L)
            di = plsc.load_gather(dinv, [ie_idx[ci, sl]])
            dj = plsc.load_gather(dinv, [je_idx[ci, sl]])
            coef_v[ci, sl] = w_v[ci, sl] * di * dj
        plsc.subcore_barrier()

        def _scale_rows(ci):
            for g in range(80 // L):
                t_c[pl.ds(0, L)] = coef_v[ci, pl.ds(g * L, L)]

                def lane_body(lane, g=g):
                    cfv = plsc.load_gather(t_c, [jnp.full((L,), lane, I32)])
                    e = g * L + lane
                    for q in range(D // L):
                        sl = pl.ds(q * L, L)
                        rows[e, sl] = rows[e, sl] * cfv

                lax.fori_loop(0, L, lambda i, _, g=g: lane_body(i, g), None)

        for ci in range(_NCH):
            pltpu.sync_copy(xt_hbm.at[je_idx.at[ci]], rows)
            _scale_rows(ci)
            pltpu.sync_copy(rows, out_sp.at[ie_idx.at[ci]], add=True)
            pltpu.sync_copy(xt_hbm.at[ie_idx.at[ci]], rows)
            _scale_rows(ci)
            pltpu.sync_copy(rows, out_sp.at[je_idx.at[ci]], add=True)
        plsc.subcore_barrier()
        pltpu.sync_copy(out_sp.at[pl.ds(s * CN, CN)],
                        outp_hbm.at[c, pl.ds(s * CN, CN)])

    return k6b(xt, degp, ie3, je3, w3)


def _k8_body(xt_ref, dg_ref, o0_ref, o1_ref, out_ref):
    d = jnp.float32(1.0) + dg_ref[:, 0:1] + dg_ref[:, 1:2]
    inv = jnp.float32(1.0) / d
    out_ref[...] = jnp.maximum(xt_ref[...] * inv + o0_ref[...] + o1_ref[...],
                               jnp.float32(0.0))


def _k8(xt, degT, o0, o1):
    return pl.pallas_call(
        _k8_body,
        grid=(_K1_GRID,),
        in_specs=[
            pl.BlockSpec((_K1_BLK, D), lambda i: (i, 0)),
            pl.BlockSpec((_K1_BLK, NC), lambda i: (i, 0)),
            pl.BlockSpec((_K1_BLK, D), lambda i: (i, 0)),
            pl.BlockSpec((_K1_BLK, D), lambda i: (i, 0)),
        ],
        out_specs=pl.BlockSpec((_K1_BLK, D), lambda i: (i, 0)),
        out_shape=jax.ShapeDtypeStruct((N, D), F32),
    )(xt, degT, o0, o1)


def kernel(x, hyperedge_index, W, b, direction):
    nodes = hyperedge_index[0]
    he = hyperedge_index[1]
    b2 = b.reshape(1, D)
    d2 = direction.reshape(1, D)
    xt, p2 = _k1(x, W, b2, d2)
    p = jnp.concatenate([p2.reshape(N), jnp.zeros((HP - N,), F32)])
    smax2, smin2 = _k2(p, nodes, he)
    ia2, ja2 = _k4(p, nodes, he, smax2, smin2)
    ie3, je3, w3, degp = _k6a(ia2, ja2)
    outp = _k6b(xt, degp, ie3, je3, w3)
    degT = jnp.stack([degp[0, :N], degp[1, :N]], axis=-1)
    out = _k8(xt, degT, outp[0, :N], outp[1, :N])
    return out


# speedup vs baseline: 27.1433x; 1.0247x over previous
import functools

import jax
import jax.numpy as jnp
from jax import lax
from jax.experimental import pallas as pl
from jax.experimental.pallas import tpu as pltpu
from jax.experimental.pallas import tpu_sc as plsc

N = 10000
H = 10000
NNZ = 320000
D = 128

NC = 2
NS = 16
NW = NC * NS
L = 16

EPT = NNZ // NW
HP = 10240
CH = HP // NS
RH = HP // NW
NP_ = 10240
CN = NP_ // NS

IMAXV = 2**31 - 1
IMINV = -(2**31)
BIG = N + 1

F32 = jnp.float32
I32 = jnp.int32


def _mesh():
    return plsc.VectorSubcoreMesh(
        core_axis_name="c", subcore_axis_name="s", num_cores=NC, num_subcores=NS
    )


_SC_PARAMS = pltpu.CompilerParams(
    use_tc_tiling_on_sc=False, needs_layout_passes=False
)


def _wid():
    return lax.axis_index("s") * NC + lax.axis_index("c")


def _iota():
    return lax.iota(I32, L)


def _sortable(pv):
    pi = plsc.bitcast(pv, I32)
    return jnp.where(pi < 0, pi ^ jnp.int32(0x7FFFFFFF), pi)


def _combine_runs(he_s, vals, ops, t_he, t_v):
    iota = _iota()
    for k in (1, 2, 4, 8):
        perm = jnp.maximum(iota - k, 0)
        t_he[pl.ds(0, L)] = he_s
        he_sh = plsc.load_gather(t_he, [perm])
        eq = (he_sh == he_s) & (iota >= k)
        new_vals = []
        for v, op in zip(vals, ops):
            t_v[pl.ds(0, L)] = v
            v_sh = plsc.load_gather(t_v, [perm])
            new_vals.append(jnp.where(eq, op(v, v_sh), v))
        vals = new_vals
    up = jnp.minimum(iota + 1, L - 1)
    t_he[pl.ds(0, L)] = he_s
    he_nx = plsc.load_gather(t_he, [up])
    is_last = (he_nx != he_s) | (iota == L - 1)
    return vals, is_last


def _sc_reduce_to_hbm(priv_refs, shared_refs, red_v, outbuf, out_hbm_parts, opfns):
    c = lax.axis_index("c")
    s = lax.axis_index("s")
    for priv, sh in zip(priv_refs, shared_refs):
        pltpu.sync_copy(priv, sh.at[s])
    plsc.subcore_barrier()
    col = s * CH
    for sh, out_hbm, opfn in zip(shared_refs, out_hbm_parts, opfns):
        for r in range(NS):
            pltpu.sync_copy(sh.at[r, pl.ds(col, CH)], red_v.at[r])

        def red_body(j, _, opfn=opfn):
            sl = pl.ds(j * L, L)
            acc = red_v[0, sl]
            for r in range(1, NS):
                acc = opfn(acc, red_v[r, sl])
            outbuf[sl] = acc
            return None

        lax.fori_loop(0, CH // L, red_body, None)
        pltpu.sync_copy(outbuf, out_hbm.at[c, pl.ds(col, CH)])


_K1_BLK = 400
_K1_GRID = N // _K1_BLK


def _k1_body(x_ref, w_ref, b_ref, d_ref, xt_ref, p_ref):
    xb = x_ref[...]
    xt = lax.dot_general(
        xb, w_ref[...], (((1,), (1,)), ((), ())),
        preferred_element_type=F32,
    )
    xt = xt + b_ref[...]
    xt_ref[...] = xt
    p_ref[...] = lax.dot_general(
        d_ref[...], xt, (((1,), (1,)), ((), ())),
        preferred_element_type=F32,
    ).reshape(1, 1, _K1_BLK)


def _k1(x, w, b2, d2):
    return pl.pallas_call(
        _k1_body,
        grid=(_K1_GRID,),
        in_specs=[
            pl.BlockSpec((_K1_BLK, D), lambda i: (i, 0)),
            pl.BlockSpec((D, D), lambda i: (0, 0)),
            pl.BlockSpec((1, D), lambda i: (0, 0)),
            pl.BlockSpec((1, D), lambda i: (0, 0)),
        ],
        out_specs=[
            pl.BlockSpec((_K1_BLK, D), lambda i: (i, 0)),
            pl.BlockSpec((1, 1, _K1_BLK), lambda i: (i, 0, 0)),
        ],
        out_shape=[
            jax.ShapeDtypeStruct((N, D), F32),
            jax.ShapeDtypeStruct((_K1_GRID, 1, _K1_BLK), F32),
        ],
    )(x, w, b2, d2)


def _k2(p, nodes, he):
    @functools.partial(
        pl.kernel,
        mesh=_mesh(),
        compiler_params=_SC_PARAMS,
        out_type=(
            jax.ShapeDtypeStruct((NC, HP), I32),
            jax.ShapeDtypeStruct((NC, HP), I32),
        ),
        scratch_types=[
            pltpu.VMEM((HP,), F32),
            pltpu.VMEM((EPT,), I32),
            pltpu.VMEM((EPT,), I32),
            pltpu.VMEM((HP,), I32),
            pltpu.VMEM((HP,), I32),
            pltpu.VMEM_SHARED((NS, HP), I32),
            pltpu.VMEM_SHARED((NS, HP), I32),
            pltpu.VMEM((NS, CH), I32),
            pltpu.VMEM((CH,), I32),
            pltpu.VMEM((128,), I32),
            pltpu.VMEM((128,), I32),
            pltpu.VMEM((HP,), I32),
        ],
    )
    def k2(p_hbm, nodes_hbm, he_hbm, smax_hbm, smin_hbm,
           p_v, nd_v, he_v, pmax, pmin, shmax, shmin, red_v, outbuf, t_he, t_v,
           t_dup):
        wid = _wid()
        base = wid * EPT
        pltpu.sync_copy(p_hbm, p_v)
        pltpu.sync_copy(nodes_hbm.at[pl.ds(base, EPT)], nd_v)
        pltpu.sync_copy(he_hbm.at[pl.ds(base, EPT)], he_v)

        def init_body(i, _):
            sl = pl.ds(i * L, L)
            pmax[sl] = jnp.full((L,), IMINV, I32)
            pmin[sl] = jnp.full((L,), IMAXV, I32)
            return None

        lax.fori_loop(0, HP // L, init_body, None)

        def body(i, _):
            sl = pl.ds(i * L, L)
            nd = nd_v[sl]
            hv = he_v[sl]
            key = _sortable(plsc.load_gather(p_v, [nd]))
            iota = _iota()
            plsc.store_scatter(t_dup, [hv], iota)
            rb = plsc.load_gather(t_dup, [hv])
            nodup = jnp.all(rb == iota)

            def fast():
                cur = plsc.load_gather(pmax, [hv])
                plsc.store_scatter(pmax, [hv], jnp.maximum(cur, key))
                cur2 = plsc.load_gather(pmin, [hv])
                plsc.store_scatter(pmin, [hv], jnp.minimum(cur2, key))

            def slow():
                he_s, key_s = plsc.sort_key_val(hv, key)
                (vmax, vmin), is_last = _combine_runs(
                    he_s, [key_s, key_s], [jnp.maximum, jnp.minimum], t_he, t_v)
                cur = plsc.load_gather(pmax, [he_s])
                plsc.store_scatter(pmax, [he_s], jnp.maximum(cur, vmax),
                                   mask=is_last)
                cur = plsc.load_gather(pmin, [he_s])
                plsc.store_scatter(pmin, [he_s], jnp.minimum(cur, vmin),
                                   mask=is_last)

            lax.cond(nodup, fast, slow)
            return None

        lax.fori_loop(0, EPT // L, body, None)

        _sc_reduce_to_hbm([pmax, pmin], [shmax, shmin], red_v, outbuf,
                          [smax_hbm, smin_hbm], [jnp.maximum, jnp.minimum])

    return k2(p, nodes, he)


def _k4(p, nodes, he, smax2, smin2):
    @functools.partial(
        pl.kernel,
        mesh=_mesh(),
        compiler_params=_SC_PARAMS,
        out_type=(
            jax.ShapeDtypeStruct((NC, HP), I32),
            jax.ShapeDtypeStruct((NC, HP), I32),
        ),
        scratch_types=[
            pltpu.VMEM((HP,), F32),
            pltpu.VMEM((EPT,), I32),
            pltpu.VMEM((EPT,), I32),
            pltpu.VMEM((HP,), I32),
            pltpu.VMEM((HP,), I32),
            pltpu.VMEM((HP,), I32),
            pltpu.VMEM((HP,), I32),
            pltpu.VMEM((HP,), I32),
            pltpu.VMEM_SHARED((NS, HP), I32),
            pltpu.VMEM_SHARED((NS, HP), I32),
            pltpu.VMEM((NS, CH), I32),
            pltpu.VMEM((CH,), I32),
            pltpu.VMEM((128,), I32),
            pltpu.VMEM((128,), I32),
            pltpu.VMEM((HP,), I32),
        ],
    )
    def k4(p_hbm, nodes_hbm, he_hbm, smax_hbm, smin_hbm, ia_hbm, ja_hbm,
           p_v, nd_v, he_v, smax, smin, tmp, pi_e, pj_e,
           shi, shj, red_v, outbuf, t_he, t_v, t_dup):
        wid = _wid()
        base = wid * EPT
        pltpu.sync_copy(p_hbm, p_v)
        pltpu.sync_copy(nodes_hbm.at[pl.ds(base, EPT)], nd_v)
        pltpu.sync_copy(he_hbm.at[pl.ds(base, EPT)], he_v)
        pltpu.sync_copy(smax_hbm.at[0], smax)
        pltpu.sync_copy(smax_hbm.at[1], tmp)

        def comb_max(i, _):
            sl = pl.ds(i * L, L)
            smax[sl] = jnp.maximum(smax[sl], tmp[sl])
            return None

        lax.fori_loop(0, HP // L, comb_max, None)
        pltpu.sync_copy(smin_hbm.at[0], smin)
        pltpu.sync_copy(smin_hbm.at[1], tmp)

        def comb_min(i, _):
            sl = pl.ds(i * L, L)
            smin[sl] = jnp.minimum(smin[sl], tmp[sl])
            pi_e[sl] = jnp.full((L,), IMAXV, I32)
            pj_e[sl] = jnp.full((L,), IMAXV, I32)
            return None

        lax.fori_loop(0, HP // L, comb_min, None)

        def body(i, _):
            sl = pl.ds(i * L, L)
            nd = nd_v[sl]
            hv = he_v[sl]
            key = _sortable(plsc.load_gather(p_v, [nd]))
            gmax = plsc.load_gather(smax, [hv])
            gmin = plsc.load_gather(smin, [hv])
            ci = jnp.where(key >= gmax, nd, BIG)
            cj = jnp.where(key <= gmin, nd, BIG)
            iota = _iota()
            plsc.store_scatter(t_dup, [hv], iota)
            rb = plsc.load_gather(t_dup, [hv])
            nodup = jnp.all(rb == iota)

            def fast():
                cur = plsc.load_gather(pi_e, [hv])
                plsc.store_scatter(pi_e, [hv], jnp.minimum(cur, ci))
                cur2 = plsc.load_gather(pj_e, [hv])
                plsc.store_scatter(pj_e, [hv], jnp.minimum(cur2, cj))

            def slow():
                he_s, ci_s = plsc.sort_key_val(hv, ci)
                _, cj_s = plsc.sort_key_val(hv, cj)
                (vi, vj), is_last = _combine_runs(
                    he_s, [ci_s, cj_s], [jnp.minimum, jnp.minimum], t_he, t_v)
                cur = plsc.load_gather(pi_e, [he_s])
                plsc.store_scatter(pi_e, [he_s], jnp.minimum(cur, vi),
                                   mask=is_last)
                cur = plsc.load_gather(pj_e, [he_s])
                plsc.store_scatter(pj_e, [he_s], jnp.minimum(cur, vj),
                                   mask=is_last)

            lax.cond(nodup, fast, slow)
            return None

        lax.fori_loop(0, EPT // L, body, None)

        _sc_reduce_to_hbm([pi_e, pj_e], [shi, shj], red_v, outbuf,
                          [ia_hbm, ja_hbm], [jnp.minimum, jnp.minimum])

    return k4(p, nodes, he, smax2, smin2)


_NCH = RH // 80


def _k6a(ia2, ja2):
    @functools.partial(
        pl.kernel,
        mesh=_mesh(),
        compiler_params=_SC_PARAMS,
        out_type=(
            jax.ShapeDtypeStruct((NW, _NCH, 80), I32),
            jax.ShapeDtypeStruct((NW, _NCH, 80), I32),
            jax.ShapeDtypeStruct((NW, _NCH, 80), F32),
            jax.ShapeDtypeStruct((NC, NP_), F32),
        ),
        scratch_types=[
            pltpu.VMEM((RH,), I32),
            pltpu.VMEM((RH,), I32),
            pltpu.VMEM((RH,), I32),
            pltpu.VMEM((RH,), I32),
            pltpu.VMEM((_NCH, 80), I32),
            pltpu.VMEM((_NCH, 80), I32),
            pltpu.VMEM((_NCH, 80), F32),
            pltpu.VMEM((CN,), F32),
            pltpu.VMEM_SHARED((NP_,), F32),
        ],
    )
    def k6a(ia_hbm, ja_hbm, ie_hbm, je_hbm, w_hbm, degp_hbm,
            a0, a1, b0, b1, ie_idx, je_idx, w_v, zbuf, deg_sp):
        c = lax.axis_index("c")
        s = lax.axis_index("s")
        wid = _wid()
        hbase = wid * RH
        pltpu.sync_copy(ia_hbm.at[0, pl.ds(hbase, RH)], a0)
        pltpu.sync_copy(ia_hbm.at[1, pl.ds(hbase, RH)], a1)
        pltpu.sync_copy(ja_hbm.at[0, pl.ds(hbase, RH)], b0)
        pltpu.sync_copy(ja_hbm.at[1, pl.ds(hbase, RH)], b1)
        for i in range(RH // L):
            sl = pl.ds(i * L, L)
            ie = jnp.minimum(a0[sl], a1[sl])
            je = jnp.minimum(b0[sl], b1[sl])
            valid = (ie < N) & (je < N)
            w = jnp.where(valid, jnp.float32(1.0), jnp.float32(0.0))
            ie = jnp.where(valid, ie, 0)
            je = jnp.where(valid, je, 0)
            ci, ro = divmod(i, 5)
            sl2 = pl.ds(ro * L, L)
            ie_idx[ci, sl2] = ie
            je_idx[ci, sl2] = je
            w_v[ci, sl2] = w
        pltpu.sync_copy(ie_idx, ie_hbm.at[wid])
        pltpu.sync_copy(je_idx, je_hbm.at[wid])
        pltpu.sync_copy(w_v, w_hbm.at[wid])

        def zero_body(i, _):
            zbuf[pl.ds(i * L, L)] = jnp.zeros((L,), F32)
            return None

        lax.fori_loop(0, CN // L, zero_body, None)
        pltpu.sync_copy(zbuf, deg_sp.at[pl.ds(s * CN, CN)])
        plsc.subcore_barrier()
        for ci in range(_NCH):
            pltpu.sync_copy(w_v.at[ci], deg_sp.at[ie_idx.at[ci]], add=True)
            pltpu.sync_copy(w_v.at[ci], deg_sp.at[je_idx.at[ci]], add=True)
        plsc.subcore_barrier()
        pltpu.sync_copy(deg_sp.at[pl.ds(s * CN, CN)],
                        degp_hbm.at[c, pl.ds(s * CN, CN)])

    return k6a(ia2, ja2)


def _k6b(xt, degp, ie3, je3, w3):
    @functools.partial(
        pl.kernel,
        mesh=_mesh(),
        compiler_params=_SC_PARAMS,
        out_type=jax.ShapeDtypeStruct((NC, NP_, D), F32),
        scratch_types=[
            pltpu.VMEM((NP_,), F32),
            pltpu.VMEM((NP_,), F32),
            pltpu.VMEM((_NCH, 80), I32),
            pltpu.VMEM((_NCH, 80), I32),
            pltpu.VMEM((_NCH, 80), F32),
            pltpu.VMEM((_NCH, 80), F32),
            pltpu.VMEM((80, D), F32),
            pltpu.VMEM((128, D), F32),
            pltpu.VMEM((128,), F32),
            pltpu.VMEM_SHARED((NP_, D), F32),
        ],
    )
    def k6b(xt_hbm, degp_hbm, ie_hbm, je_hbm, w_hbm, outp_hbm,
            tmp, dinv, ie_idx, je_idx, w_v, coef_v, rows, zblk, t_c, out_sp):
        c = lax.axis_index("c")
        s = lax.axis_index("s")
        wid = _wid()

        def zfill(i, _):
            for q in range(D // L):
                zblk[i, pl.ds(q * L, L)] = jnp.zeros((L,), F32)
            return None

        lax.fori_loop(0, 128, zfill, None)
        for kk in range(CN // 128):
            pltpu.sync_copy(zblk, out_sp.at[pl.ds(s * CN + kk * 128, 128)])

        pltpu.sync_copy(degp_hbm.at[0], tmp)
        pltpu.sync_copy(degp_hbm.at[1], dinv)

        def dinv_body(i, _):
            sl = pl.ds(i * L, L)
            d = tmp[sl] + dinv[sl] + jnp.float32(1.0)
            bi = plsc.bitcast(d, I32)
            y = plsc.bitcast(jnp.int32(0x5F3759DF) - (bi >> 1), F32)
            half_d = jnp.float32(0.5) * d
            for _u in range(3):
                y = y * (jnp.float32(1.5) - half_d * y * y)
            dinv[sl] = y
            return None

        lax.fori_loop(0, NP_ // L, dinv_body, None)

        pltpu.sync_copy(ie_hbm.at[wid], ie_idx)
        pltpu.sync_copy(je_hbm.at[wid], je_idx)
        pltpu.sync_copy(w_hbm.at[wid], w_v)
        for i in range(RH // L):
            ci, ro = divmod(i, 5)
            sl = pl.ds(ro * L, L)
            di = plsc.load_gather(dinv, [ie_idx[ci, sl]])
            dj = plsc.load_gather(dinv, [je_idx[ci, sl]])
            coef_v[ci, sl] = w_v[ci, sl] * di * dj
        plsc.subcore_barrier()

        def _scale_rows(ci):
            for g in range(80 // L):
                t_c[pl.ds(0, L)] = coef_v[ci, pl.ds(g * L, L)]

                def lane_body(lane, g=g):
                    cfv = plsc.load_gather(t_c, [jnp.full((L,), lane, I32)])
                    e = g * L + lane
                    for q in range(D // L):
                        sl = pl.ds(q * L, L)
                        rows[e, sl] = rows[e, sl] * cfv

                lax.fori_loop(0, L, lambda i, _, g=g: lane_body(i, g), None)

        for ci in range(_NCH):
            pltpu.sync_copy(xt_hbm.at[je_idx.at[ci]], rows)
            _scale_rows(ci)
            pltpu.sync_copy(rows, out_sp.at[ie_idx.at[ci]], add=True)
            pltpu.sync_copy(xt_hbm.at[ie_idx.at[ci]], rows)
            _scale_rows(ci)
            pltpu.sync_copy(rows, out_sp.at[je_idx.at[ci]], add=True)
        plsc.subcore_barrier()
        pltpu.sync_copy(out_sp.at[pl.ds(s * CN, CN)],
                        outp_hbm.at[c, pl.ds(s * CN, CN)])

    return k6b(xt, degp, ie3, je3, w3)


def _k8_body(xt_ref, dg_ref, o0_ref, o1_ref, out_ref):
    d = jnp.float32(1.0) + dg_ref[:, 0:1] + dg_ref[:, 1:2]
    inv = jnp.float32(1.0) / d
    out_ref[...] = jnp.maximum(xt_ref[...] * inv + o0_ref[...] + o1_ref[...],
                               jnp.float32(0.0))


def _k8(xt, degT, o0, o1):
    return pl.pallas_call(
        _k8_body,
        grid=(_K1_GRID,),
        in_specs=[
            pl.BlockSpec((_K1_BLK, D), lambda i: (i, 0)),
            pl.BlockSpec((_K1_BLK, NC), lambda i: (i, 0)),
            pl.BlockSpec((_K1_BLK, D), lambda i: (i, 0)),
            pl.BlockSpec((_K1_BLK, D), lambda i: (i, 0)),
        ],
        out_specs=pl.BlockSpec((_K1_BLK, D), lambda i: (i, 0)),
        out_shape=jax.ShapeDtypeStruct((N, D), F32),
    )(xt, degT, o0, o1)


def kernel(x, hyperedge_index, W, b, direction):
    nodes = hyperedge_index[0]
    he = hyperedge_index[1]
    b2 = b.reshape(1, D)
    d2 = direction.reshape(1, D)
    xt, p2 = _k1(x, W, b2, d2)
    p = jnp.concatenate([p2.reshape(N), jnp.zeros((HP - N,), F32)])
    smax2, smin2 = _k2(p, nodes, he)
    ia2, ja2 = _k4(p, nodes, he, smax2, smin2)
    ie3, je3, w3, degp = _k6a(ia2, ja2)
    outp = _k6b(xt, degp, ie3, je3, w3)
    degT = jnp.stack([degp[0, :N], degp[1, :N]], axis=-1)
    out = _k8(xt, degT, outp[0, :N], outp[1, :N])
    return out


# speedup vs baseline: 28.0378x; 1.0330x over previous
import functools

import jax
import jax.numpy as jnp
from jax import lax
from jax.experimental import pallas as pl
from jax.experimental.pallas import tpu as pltpu
from jax.experimental.pallas import tpu_sc as plsc

N = 10000
H = 10000
NNZ = 320000
D = 128

NC = 2
NS = 16
NW = NC * NS
L = 16

EPT = NNZ // NW
HP = 10240
CH = HP // NS
RH = HP // NW
NP_ = 10240
CN = NP_ // NS

IMAXV = 2**31 - 1
IMINV = -(2**31)
BIG = N + 1

F32 = jnp.float32
I32 = jnp.int32


def _mesh():
    return plsc.VectorSubcoreMesh(
        core_axis_name="c", subcore_axis_name="s", num_cores=NC, num_subcores=NS
    )


_SC_PARAMS = pltpu.CompilerParams(
    use_tc_tiling_on_sc=False, needs_layout_passes=False
)


def _wid():
    return lax.axis_index("s") * NC + lax.axis_index("c")


def _iota():
    return lax.iota(I32, L)


def _sortable(pv):
    pi = plsc.bitcast(pv, I32)
    return jnp.where(pi < 0, pi ^ jnp.int32(0x7FFFFFFF), pi)


def _combine_runs(he_s, vals, ops, t_he, t_v):
    iota = _iota()
    for k in (1, 2, 4, 8):
        perm = jnp.maximum(iota - k, 0)
        t_he[pl.ds(0, L)] = he_s
        he_sh = plsc.load_gather(t_he, [perm])
        eq = (he_sh == he_s) & (iota >= k)
        new_vals = []
        for v, op in zip(vals, ops):
            t_v[pl.ds(0, L)] = v
            v_sh = plsc.load_gather(t_v, [perm])
            new_vals.append(jnp.where(eq, op(v, v_sh), v))
        vals = new_vals
    up = jnp.minimum(iota + 1, L - 1)
    t_he[pl.ds(0, L)] = he_s
    he_nx = plsc.load_gather(t_he, [up])
    is_last = (he_nx != he_s) | (iota == L - 1)
    return vals, is_last


def _sc_reduce_to_hbm(priv_refs, shared_refs, red_v, outbuf, out_hbm_parts, opfns):
    c = lax.axis_index("c")
    s = lax.axis_index("s")
    for priv, sh in zip(priv_refs, shared_refs):
        pltpu.sync_copy(priv, sh.at[s])
    plsc.subcore_barrier()
    col = s * CH
    for sh, out_hbm, opfn in zip(shared_refs, out_hbm_parts, opfns):
        for r in range(NS):
            pltpu.sync_copy(sh.at[r, pl.ds(col, CH)], red_v.at[r])

        def red_body(j, _, opfn=opfn):
            sl = pl.ds(j * L, L)
            acc = red_v[0, sl]
            for r in range(1, NS):
                acc = opfn(acc, red_v[r, sl])
            outbuf[sl] = acc
            return None

        lax.fori_loop(0, CH // L, red_body, None)
        pltpu.sync_copy(outbuf, out_hbm.at[c, pl.ds(col, CH)])


_K1_BLK = 400
_K1_GRID = N // _K1_BLK


def _k1_body(x_ref, w_ref, b_ref, d_ref, xt_ref, p_ref):
    xb = x_ref[...]
    xt = lax.dot_general(
        xb, w_ref[...], (((1,), (1,)), ((), ())),
        preferred_element_type=F32,
    )
    xt = xt + b_ref[...]
    xt_ref[...] = xt
    p_ref[...] = lax.dot_general(
        d_ref[...], xt, (((1,), (1,)), ((), ())),
        preferred_element_type=F32,
    ).reshape(1, 1, _K1_BLK)


def _k1(x, w, b2, d2):
    return pl.pallas_call(
        _k1_body,
        grid=(_K1_GRID,),
        in_specs=[
            pl.BlockSpec((_K1_BLK, D), lambda i: (i, 0)),
            pl.BlockSpec((D, D), lambda i: (0, 0)),
            pl.BlockSpec((1, D), lambda i: (0, 0)),
            pl.BlockSpec((1, D), lambda i: (0, 0)),
        ],
        out_specs=[
            pl.BlockSpec((_K1_BLK, D), lambda i: (i, 0)),
            pl.BlockSpec((1, 1, _K1_BLK), lambda i: (i, 0, 0)),
        ],
        out_shape=[
            jax.ShapeDtypeStruct((N, D), F32),
            jax.ShapeDtypeStruct((_K1_GRID, 1, _K1_BLK), F32),
        ],
    )(x, w, b2, d2)


def _k2(p, nodes, he):
    @functools.partial(
        pl.kernel,
        mesh=_mesh(),
        compiler_params=_SC_PARAMS,
        out_type=(
            jax.ShapeDtypeStruct((NC, HP), I32),
            jax.ShapeDtypeStruct((NC, HP), I32),
        ),
        scratch_types=[
            pltpu.VMEM((HP,), F32),
            pltpu.VMEM((EPT,), I32),
            pltpu.VMEM((EPT,), I32),
            pltpu.VMEM((HP,), I32),
            pltpu.VMEM((HP,), I32),
            pltpu.VMEM_SHARED((NS, HP), I32),
            pltpu.VMEM_SHARED((NS, HP), I32),
            pltpu.VMEM((NS, CH), I32),
            pltpu.VMEM((CH,), I32),
            pltpu.VMEM((128,), I32),
            pltpu.VMEM((128,), I32),
            pltpu.VMEM((HP,), I32),
        ],
    )
    def k2(p_hbm, nodes_hbm, he_hbm, smax_hbm, smin_hbm,
           p_v, nd_v, he_v, pmax, pmin, shmax, shmin, red_v, outbuf, t_he, t_v,
           t_dup):
        wid = _wid()
        base = wid * EPT
        pltpu.sync_copy(p_hbm, p_v)
        pltpu.sync_copy(nodes_hbm.at[pl.ds(base, EPT)], nd_v)
        pltpu.sync_copy(he_hbm.at[pl.ds(base, EPT)], he_v)

        def init_body(i, _):
            sl = pl.ds(i * L, L)
            pmax[sl] = jnp.full((L,), IMINV, I32)
            pmin[sl] = jnp.full((L,), IMAXV, I32)
            return None

        lax.fori_loop(0, HP // L, init_body, None)

        def body(i, _):
            sl = pl.ds(i * L, L)
            nd = nd_v[sl]
            hv = he_v[sl]
            key = _sortable(plsc.load_gather(p_v, [nd]))
            iota = _iota()
            plsc.store_scatter(t_dup, [hv], iota)
            rb = plsc.load_gather(t_dup, [hv])
            nodup = jnp.all(rb == iota)

            def fast():
                cur = plsc.load_gather(pmax, [hv])
                plsc.store_scatter(pmax, [hv], jnp.maximum(cur, key))
                cur2 = plsc.load_gather(pmin, [hv])
                plsc.store_scatter(pmin, [hv], jnp.minimum(cur2, key))

            def slow():
                he_s, key_s = plsc.sort_key_val(hv, key)
                (vmax, vmin), is_last = _combine_runs(
                    he_s, [key_s, key_s], [jnp.maximum, jnp.minimum], t_he, t_v)
                cur = plsc.load_gather(pmax, [he_s])
                plsc.store_scatter(pmax, [he_s], jnp.maximum(cur, vmax),
                                   mask=is_last)
                cur = plsc.load_gather(pmin, [he_s])
                plsc.store_scatter(pmin, [he_s], jnp.minimum(cur, vmin),
                                   mask=is_last)

            lax.cond(nodup, fast, slow)
            return None

        lax.fori_loop(0, EPT // L, body, None)

        _sc_reduce_to_hbm([pmax, pmin], [shmax, shmin], red_v, outbuf,
                          [smax_hbm, smin_hbm], [jnp.maximum, jnp.minimum])

    return k2(p, nodes, he)


def _k4(p, nodes, he, smax2, smin2):
    @functools.partial(
        pl.kernel,
        mesh=_mesh(),
        compiler_params=_SC_PARAMS,
        out_type=(
            jax.ShapeDtypeStruct((NC, HP), I32),
            jax.ShapeDtypeStruct((NC, HP), I32),
        ),
        scratch_types=[
            pltpu.VMEM((HP,), F32),
            pltpu.VMEM((EPT,), I32),
            pltpu.VMEM((EPT,), I32),
            pltpu.VMEM((HP,), I32),
            pltpu.VMEM((HP,), I32),
            pltpu.VMEM((HP,), I32),
            pltpu.VMEM((HP,), I32),
            pltpu.VMEM((HP,), I32),
            pltpu.VMEM_SHARED((NS, HP), I32),
            pltpu.VMEM_SHARED((NS, HP), I32),
            pltpu.VMEM((NS, CH), I32),
            pltpu.VMEM((CH,), I32),
            pltpu.VMEM((128,), I32),
            pltpu.VMEM((128,), I32),
            pltpu.VMEM((HP,), I32),
        ],
    )
    def k4(p_hbm, nodes_hbm, he_hbm, smax_hbm, smin_hbm, ia_hbm, ja_hbm,
           p_v, nd_v, he_v, smax, smin, tmp, pi_e, pj_e,
           shi, shj, red_v, outbuf, t_he, t_v, t_dup):
        wid = _wid()
        base = wid * EPT
        pltpu.sync_copy(p_hbm, p_v)
        pltpu.sync_copy(nodes_hbm.at[pl.ds(base, EPT)], nd_v)
        pltpu.sync_copy(he_hbm.at[pl.ds(base, EPT)], he_v)
        pltpu.sync_copy(smax_hbm.at[0], smax)
        pltpu.sync_copy(smax_hbm.at[1], tmp)

        def comb_max(i, _):
            sl = pl.ds(i * L, L)
            smax[sl] = jnp.maximum(smax[sl], tmp[sl])
            return None

        lax.fori_loop(0, HP // L, comb_max, None)
        pltpu.sync_copy(smin_hbm.at[0], smin)
        pltpu.sync_copy(smin_hbm.at[1], tmp)

        def comb_min(i, _):
            sl = pl.ds(i * L, L)
            smin[sl] = jnp.minimum(smin[sl], tmp[sl])
            pi_e[sl] = jnp.full((L,), IMAXV, I32)
            pj_e[sl] = jnp.full((L,), IMAXV, I32)
            return None

        lax.fori_loop(0, HP // L, comb_min, None)

        def body(i, _):
            sl = pl.ds(i * L, L)
            nd = nd_v[sl]
            hv = he_v[sl]
            key = _sortable(plsc.load_gather(p_v, [nd]))
            gmax = plsc.load_gather(smax, [hv])
            gmin = plsc.load_gather(smin, [hv])
            ci = jnp.where(key >= gmax, nd, BIG)
            cj = jnp.where(key <= gmin, nd, BIG)
            iota = _iota()
            plsc.store_scatter(t_dup, [hv], iota)
            rb = plsc.load_gather(t_dup, [hv])
            nodup = jnp.all(rb == iota)

            def fast():
                cur = plsc.load_gather(pi_e, [hv])
                plsc.store_scatter(pi_e, [hv], jnp.minimum(cur, ci))
                cur2 = plsc.load_gather(pj_e, [hv])
                plsc.store_scatter(pj_e, [hv], jnp.minimum(cur2, cj))

            def slow():
                he_s, ci_s = plsc.sort_key_val(hv, ci)
                _, cj_s = plsc.sort_key_val(hv, cj)
                (vi, vj), is_last = _combine_runs(
                    he_s, [ci_s, cj_s], [jnp.minimum, jnp.minimum], t_he, t_v)
                cur = plsc.load_gather(pi_e, [he_s])
                plsc.store_scatter(pi_e, [he_s], jnp.minimum(cur, vi),
                                   mask=is_last)
                cur = plsc.load_gather(pj_e, [he_s])
                plsc.store_scatter(pj_e, [he_s], jnp.minimum(cur, vj),
                                   mask=is_last)

            lax.cond(nodup, fast, slow)
            return None

        lax.fori_loop(0, EPT // L, body, None)

        _sc_reduce_to_hbm([pi_e, pj_e], [shi, shj], red_v, outbuf,
                          [ia_hbm, ja_hbm], [jnp.minimum, jnp.minimum])

    return k4(p, nodes, he, smax2, smin2)


_NCH = RH // 80


def _k6a(ia2, ja2):
    @functools.partial(
        pl.kernel,
        mesh=_mesh(),
        compiler_params=_SC_PARAMS,
        out_type=(
            jax.ShapeDtypeStruct((NW, _NCH, 80), I32),
            jax.ShapeDtypeStruct((NW, _NCH, 80), I32),
            jax.ShapeDtypeStruct((NW, _NCH, 80), F32),
            jax.ShapeDtypeStruct((NC, NP_), F32),
        ),
        scratch_types=[
            pltpu.VMEM((RH,), I32),
            pltpu.VMEM((RH,), I32),
            pltpu.VMEM((RH,), I32),
            pltpu.VMEM((RH,), I32),
            pltpu.VMEM((_NCH, 80), I32),
            pltpu.VMEM((_NCH, 80), I32),
            pltpu.VMEM((_NCH, 80), F32),
            pltpu.VMEM((CN,), F32),
            pltpu.VMEM_SHARED((NP_,), F32),
        ],
    )
    def k6a(ia_hbm, ja_hbm, ie_hbm, je_hbm, w_hbm, degp_hbm,
            a0, a1, b0, b1, ie_idx, je_idx, w_v, zbuf, deg_sp):
        c = lax.axis_index("c")
        s = lax.axis_index("s")
        wid = _wid()
        hbase = wid * RH
        pltpu.sync_copy(ia_hbm.at[0, pl.ds(hbase, RH)], a0)
        pltpu.sync_copy(ia_hbm.at[1, pl.ds(hbase, RH)], a1)
        pltpu.sync_copy(ja_hbm.at[0, pl.ds(hbase, RH)], b0)
        pltpu.sync_copy(ja_hbm.at[1, pl.ds(hbase, RH)], b1)
        for i in range(RH // L):
            sl = pl.ds(i * L, L)
            ie = jnp.minimum(a0[sl], a1[sl])
            je = jnp.minimum(b0[sl], b1[sl])
            valid = (ie < N) & (je < N)
            w = jnp.where(valid, jnp.float32(1.0), jnp.float32(0.0))
            ie = jnp.where(valid, ie, 0)
            je = jnp.where(valid, je, 0)
            ci, ro = divmod(i, 5)
            sl2 = pl.ds(ro * L, L)
            ie_idx[ci, sl2] = ie
            je_idx[ci, sl2] = je
            w_v[ci, sl2] = w
        pltpu.sync_copy(ie_idx, ie_hbm.at[wid])
        pltpu.sync_copy(je_idx, je_hbm.at[wid])
        pltpu.sync_copy(w_v, w_hbm.at[wid])

        def zero_body(i, _):
            zbuf[pl.ds(i * L, L)] = jnp.zeros((L,), F32)
            return None

        lax.fori_loop(0, CN // L, zero_body, None)
        pltpu.sync_copy(zbuf, deg_sp.at[pl.ds(s * CN, CN)])
        plsc.subcore_barrier()
        for ci in range(_NCH):
            pltpu.sync_copy(w_v.at[ci], deg_sp.at[ie_idx.at[ci]], add=True)
            pltpu.sync_copy(w_v.at[ci], deg_sp.at[je_idx.at[ci]], add=True)
        plsc.subcore_barrier()
        pltpu.sync_copy(deg_sp.at[pl.ds(s * CN, CN)],
                        degp_hbm.at[c, pl.ds(s * CN, CN)])

    return k6a(ia2, ja2)


def _k6b(xt, degp, ie3, je3, w3):
    @functools.partial(
        pl.kernel,
        mesh=_mesh(),
        compiler_params=_SC_PARAMS,
        out_type=jax.ShapeDtypeStruct((NC, NP_, D), F32),
        scratch_types=[
            pltpu.VMEM((2048,), F32),
            pltpu.VMEM((NP_,), F32),
            pltpu.VMEM((_NCH, 80), I32),
            pltpu.VMEM((_NCH, 80), I32),
            pltpu.VMEM((_NCH, 80), F32),
            pltpu.VMEM((_NCH, 80), F32),
            pltpu.VMEM((80, D), F32),
            pltpu.VMEM((80, D), F32),
            pltpu.VMEM((64, D), F32),
            pltpu.VMEM((128,), F32),
            pltpu.VMEM_SHARED((NP_, D), F32),
            pltpu.SemaphoreType.DMA,
            pltpu.SemaphoreType.DMA,
        ],
    )
    def k6b(xt_hbm, degp_hbm, ie_hbm, je_hbm, w_hbm, outp_hbm,
            stage, dinv, ie_idx, je_idx, w_v, coef_v, rows_a, rows_b, zblk, t_c,
            out_sp, sem_a, sem_b):
        c = lax.axis_index("c")
        s = lax.axis_index("s")
        wid = _wid()

        def zfill(i, _):
            for q in range(D // L):
                zblk[i, pl.ds(q * L, L)] = jnp.zeros((L,), F32)
            return None

        lax.fori_loop(0, 64, zfill, None)
        for kk in range(CN // 64):
            pltpu.sync_copy(zblk, out_sp.at[pl.ds(s * CN + kk * 64, 64)])

        pltpu.sync_copy(degp_hbm.at[1], dinv)
        for blk in range(NP_ // 2048):
            pltpu.sync_copy(degp_hbm.at[0, pl.ds(blk * 2048, 2048)], stage)

            def dinv_body(i, _, blk=blk):
                sl = pl.ds(i * L, L)
                gl = pl.ds(blk * 2048 + i * L, L)
                d = stage[sl] + dinv[gl] + jnp.float32(1.0)
                bi = plsc.bitcast(d, I32)
                y = plsc.bitcast(jnp.int32(0x5F3759DF) - (bi >> 1), F32)
                half_d = jnp.float32(0.5) * d
                for _u in range(3):
                    y = y * (jnp.float32(1.5) - half_d * y * y)
                dinv[gl] = y
                return None

            lax.fori_loop(0, 2048 // L, dinv_body, None)

        pltpu.sync_copy(ie_hbm.at[wid], ie_idx)
        pltpu.sync_copy(je_hbm.at[wid], je_idx)
        pltpu.sync_copy(w_hbm.at[wid], w_v)
        for i in range(RH // L):
            ci, ro = divmod(i, 5)
            sl = pl.ds(ro * L, L)
            di = plsc.load_gather(dinv, [ie_idx[ci, sl]])
            dj = plsc.load_gather(dinv, [je_idx[ci, sl]])
            coef_v[ci, sl] = w_v[ci, sl] * di * dj
        plsc.subcore_barrier()

        def _scale_rows(rows, ci):
            for g in range(80 // L):
                t_c[pl.ds(0, L)] = coef_v[ci, pl.ds(g * L, L)]

                def lane_body(lane, g=g, rows=rows):
                    cfv = plsc.load_gather(t_c, [jnp.full((L,), lane, I32)])
                    e = g * L + lane
                    for q in range(D // L):
                        sl = pl.ds(q * L, L)
                        rows[e, sl] = rows[e, sl] * cfv

                lax.fori_loop(0, L, lambda i, _, g=g: lane_body(i, g), None)

        steps = []
        for ci in range(_NCH):
            steps.append((je_idx.at[ci], ie_idx.at[ci], ci))
            steps.append((ie_idx.at[ci], je_idx.at[ci], ci))
        bufs = (rows_a, rows_b)
        sems = (sem_a, sem_b)
        descs = [None, None]
        descs[0] = pltpu.async_copy(xt_hbm.at[steps[0][0]], bufs[0], sems[0])
        for k, (src_idx, dst_idx, ci) in enumerate(steps):
            nb = (k + 1) % 2
            if k + 1 < len(steps):
                descs[nb] = pltpu.async_copy(
                    xt_hbm.at[steps[k + 1][0]], bufs[nb], sems[nb])
            descs[k % 2].wait()
            _scale_rows(bufs[k % 2], ci)
            pltpu.sync_copy(bufs[k % 2], out_sp.at[dst_idx], add=True)
        plsc.subcore_barrier()
        pltpu.sync_copy(out_sp.at[pl.ds(s * CN, CN)],
                        outp_hbm.at[c, pl.ds(s * CN, CN)])

    return k6b(xt, degp, ie3, je3, w3)


def _k8_body(xt_ref, dg_ref, o0_ref, o1_ref, out_ref):
    d = jnp.float32(1.0) + dg_ref[:, 0:1] + dg_ref[:, 1:2]
    inv = jnp.float32(1.0) / d
    out_ref[...] = jnp.maximum(xt_ref[...] * inv + o0_ref[...] + o1_ref[...],
                               jnp.float32(0.0))


def _k8(xt, degT, o0, o1):
    return pl.pallas_call(
        _k8_body,
        grid=(_K1_GRID,),
        in_specs=[
            pl.BlockSpec((_K1_BLK, D), lambda i: (i, 0)),
            pl.BlockSpec((_K1_BLK, NC), lambda i: (i, 0)),
            pl.BlockSpec((_K1_BLK, D), lambda i: (i, 0)),
            pl.BlockSpec((_K1_BLK, D), lambda i: (i, 0)),
        ],
        out_specs=pl.BlockSpec((_K1_BLK, D), lambda i: (i, 0)),
        out_shape=jax.ShapeDtypeStruct((N, D), F32),
    )(xt, degT, o0, o1)


def kernel(x, hyperedge_index, W, b, direction):
    nodes = hyperedge_index[0]
    he = hyperedge_index[1]
    b2 = b.reshape(1, D)
    d2 = direction.reshape(1, D)
    xt, p2 = _k1(x, W, b2, d2)
    p = jnp.concatenate([p2.reshape(N), jnp.zeros((HP - N,), F32)])
    smax2, smin2 = _k2(p, nodes, he)
    ia2, ja2 = _k4(p, nodes, he, smax2, smin2)
    ie3, je3, w3, degp = _k6a(ia2, ja2)
    outp = _k6b(xt, degp, ie3, je3, w3)
    degT = jnp.stack([degp[0, :N], degp[1, :N]], axis=-1)
    out = _k8(xt, degT, outp[0, :N], outp[1, :N])
    return out
